# baseline (speedup 1.0000x reference)
import functools

import jax
import jax.numpy as jnp
from jax import lax
from jax.experimental import pallas as pl
from jax.experimental.pallas import tpu as pltpu
from jax.experimental.pallas import tpu_sc as plsc

_D = 64
_NC = 2
_NS = 16
_NW = _NC * _NS
_CHUNK = 128


@functools.lru_cache(maxsize=None)
def _make_gather(b_total: int):
    b_per_w = b_total // _NW
    n_chunks = b_per_w // _CHUNK
    mesh = plsc.VectorSubcoreMesh(core_axis_name="c", subcore_axis_name="s")

    @functools.partial(
        pl.kernel,
        mesh=mesh,
        out_type=jax.ShapeDtypeStruct((_NW, n_chunks, _CHUNK, _D), jnp.float32),
        scratch_types=[
            pltpu.VMEM((n_chunks, _CHUNK), jnp.int32),
            pltpu.VMEM((n_chunks, _CHUNK, _D), jnp.float32),
            pltpu.SemaphoreType.DMA,
        ],
        compiler_params=pltpu.CompilerParams(use_tc_tiling_on_sc=False),
    )
    def gather(x_hbm, w_hbm, out_hbm, idx_v, rows_v, sem):
        wid = lax.axis_index("s") * _NC + lax.axis_index("c")
        pltpu.sync_copy(x_hbm.at[wid], idx_v)
        copies = [
            pltpu.async_copy(w_hbm.at[idx_v.at[j]], rows_v.at[j], sem)
            for j in range(n_chunks)
        ]
        for c in copies:
            c.wait()
        pltpu.sync_copy(rows_v, out_hbm.at[wid])

    return gather


def kernel(x, weight):
    b0, b1 = x.shape
    b_total = b0 * b1
    xr = x.reshape(_NW, b_total // _NW // _CHUNK, _CHUNK).astype(jnp.int32)
    out = _make_gather(b_total)(xr, weight)
    return out.reshape(b0, b1, _D)


# speedup vs baseline: 1.9266x; 1.9266x over previous
import functools

import jax
import jax.numpy as jnp
from jax import lax
from jax.experimental import pallas as pl
from jax.experimental.pallas import tpu as pltpu
from jax.experimental.pallas import tpu_sc as plsc

_D = 64
_PITCH = 128
_NC = 2
_NS = 16
_NW = _NC * _NS
_BPW = 1024
_K = 32
_NCHUNK = _BPW // _K


@functools.lru_cache(maxsize=None)
def _make_gather(b0: int, b1: int):
    wpb = _NW // b0
    bpb = b1 // wpb
    mesh = plsc.VectorSubcoreMesh(core_axis_name="c", subcore_axis_name="s")

    @functools.partial(
        pl.kernel,
        mesh=mesh,
        out_type=jax.ShapeDtypeStruct((b0 * b1, _PITCH), jnp.float32),
        scratch_types=[
            pltpu.VMEM((_BPW,), jnp.int32),
            pltpu.VMEM((_K, 8, _D), jnp.float32),
            pltpu.VMEM((_K, _PITCH), jnp.float32),
            pltpu.SemaphoreType.DMA,
        ],
        compiler_params=pltpu.CompilerParams(use_tc_tiling_on_sc=True),
    )
    def gather(x_hbm, w_hbm, out_hbm, idx_v, tiles_v, rows_v, sem):
        wid = lax.axis_index("s") * _NC + lax.axis_index("c")
        b = wid // wpb
        off = (wid % wpb) * bpb
        for k in range(_BPW // _PITCH):
            pltpu.sync_copy(
                x_hbm.at[b, pl.ds(off + k * _PITCH, _PITCH)],
                idx_v.at[pl.ds(k * _PITCH, _PITCH)],
            )
        rowbase = b * b1 + off

        def chunk_body(c, carry):
            copies = []
            vecs = []
            for j16 in range(_K // 16):
                v = idx_v[pl.ds(c * _K + j16 * 16, 16)]
                vecs.append(v)
                for l in range(16):
                    j = j16 * 16 + l
                    copies.append(
                        pltpu.async_copy(
                            w_hbm.at[v[l] >> 3], tiles_v.at[j], sem
                        )
                    )
            for h in copies:
                h.wait()
            for j16 in range(_K // 16):
                v = vecs[j16]
                for l in range(16):
                    j = j16 * 16 + l
                    r = v[l] & 7
                    for cc in range(_D // 16):
                        rows_v[j, pl.ds(cc * 16, 16)] = tiles_v[
                            j, r, pl.ds(cc * 16, 16)
                        ]
            pltpu.sync_copy(rows_v, out_hbm.at[pl.ds(rowbase + c * _K, _K)])
            return carry

        lax.fori_loop(0, _NCHUNK, chunk_body, 0)

    return gather


def kernel(x, weight):
    b0, b1 = x.shape
    w3 = weight.reshape(weight.shape[0] // 8, 8, _D)
    out2 = _make_gather(b0, b1)(x.astype(jnp.int32), w3)
    return out2[:, :_D].reshape(b0, b1, _D)
